# baseline (speedup 1.0000x reference)
import functools

import jax
import jax.numpy as jnp
from jax import lax
from jax.experimental import pallas as pl
from jax.experimental.pallas import tpu as pltpu
from jax.experimental.pallas import tpu_sc as plsc

_NUM_CORES = 2
_NUM_SUBCORES = 16
_NW = _NUM_CORES * _NUM_SUBCORES


@functools.partial(jax.jit, static_argnames=("s", "b", "d"))
def _sc_gather(idx2d, table, s, b, d):
    w_cols = b // _NW
    mesh = plsc.VectorSubcoreMesh(core_axis_name="c", subcore_axis_name="s")

    @functools.partial(
        pl.kernel,
        mesh=mesh,
        out_type=jax.ShapeDtypeStruct((s, b, d), jnp.float32),
        scratch_types=[
            pltpu.VMEM((s, w_cols), jnp.int32),
            pltpu.VMEM((2, w_cols, d), jnp.float32),
            pltpu.SemaphoreType.DMA,
            pltpu.SemaphoreType.DMA,
            pltpu.SemaphoreType.DMA,
            pltpu.SemaphoreType.DMA,
        ],
        compiler_params=pltpu.CompilerParams(use_tc_tiling_on_sc=False),
    )
    def k(idx_hbm, table_hbm, out_hbm, idx_v, rows_v, g0, g1, o0, o1):
        wid = lax.axis_index("s") * _NUM_CORES + lax.axis_index("c")
        base = wid * w_cols
        gat = (g0, g1)
        out = (o0, o1)

        def wait_gather(bb):
            pltpu.make_async_copy(
                table_hbm.at[pl.ds(0, w_cols)], rows_v.at[bb], gat[bb]
            ).wait()

        def wait_write(bb):
            pltpu.make_async_copy(
                rows_v.at[bb], out_hbm.at[0, pl.ds(base, w_cols)], out[bb]
            ).wait()

        def gather(row, bb):
            pltpu.async_copy(table_hbm.at[idx_v.at[row]], rows_v.at[bb], gat[bb])

        def write(row, bb):
            pltpu.async_copy(
                rows_v.at[bb], out_hbm.at[row, pl.ds(base, w_cols)], out[bb]
            )

        pltpu.sync_copy(idx_hbm.at[:, pl.ds(base, w_cols)], idx_v)
        gather(0, 0)

        def body(i, _):
            @pl.when(i > 0)
            def _():
                wait_write(1)
            gather(2 * i + 1, 1)
            wait_gather(0)
            write(2 * i, 0)
            wait_write(0)
            @pl.when(i < s // 2 - 1)
            def _():
                gather(2 * i + 2, 0)
            wait_gather(1)
            write(2 * i + 1, 1)
            return _

        lax.fori_loop(0, s // 2, body, None)
        wait_write(1)

    return k(idx2d, table)


def kernel(inputs, table):
    n, s = inputs.shape
    d = table.shape[1]
    idx2d = jnp.swapaxes(inputs, 0, 1).astype(jnp.int32)
    out = _sc_gather(idx2d, table, s, n, d)
    return out.transpose(1, 0, 2)
